# baseline (speedup 1.0000x reference)
import functools

import jax
import jax.numpy as jnp
from jax import lax
from jax.experimental import pallas as pl
from jax.experimental.pallas import tpu as pltpu
from jax.experimental.pallas import tpu_sc as plsc

N = 320000
D = 128
S = 10000
C = 128
NCHUNK = N // C
NC = 2
NS = 16
NW = NC * NS
ITERS = (NCHUNK + NW - 1) // NW
SLAB = 632
SLAB_LAST = S - (NS - 1) * SLAB
ZROWS = 128


def _sc_partials(x, batch):
    mesh = plsc.VectorSubcoreMesh(core_axis_name="c", subcore_axis_name="s")

    @functools.partial(
        pl.kernel,
        mesh=mesh,
        out_type=jax.ShapeDtypeStruct((NC, S, D), jnp.float32),
        scratch_types=[
            pltpu.VMEM((C, D), jnp.float32),
            pltpu.VMEM((C, D), jnp.float32),
            pltpu.VMEM((C,), jnp.int32),
            pltpu.VMEM((C,), jnp.int32),
            pltpu.VMEM((ZROWS, D), jnp.float32),
            pltpu.VMEM_SHARED((S, D), jnp.float32),
            pltpu.SemaphoreType.DMA,
            pltpu.SemaphoreType.DMA,
            pltpu.SemaphoreType.DMA,
            pltpu.SemaphoreType.DMA,
        ],
    )
    def k(x_hbm, b_hbm, out_hbm, rows_a, rows_b, idx_a, idx_b, zbuf_v,
          acc_sh, sem_a, sem_b, ssem_a, ssem_b):
        cid = lax.axis_index("c")
        sid = lax.axis_index("s")
        wid = sid * NC + cid

        zeros16 = jnp.zeros((16,), jnp.float32)

        def zbody(i, carry):
            zbuf_v[i // 8, pl.ds((i % 8) * 16, 16)] = zeros16
            return carry

        lax.fori_loop(0, ZROWS * 8, zbody, 0)

        seg_base = pl.multiple_of(sid * SLAB, 8)

        def init_slab(nrows):
            full, rem = divmod(nrows, ZROWS)
            for kk in range(full):
                pltpu.sync_copy(
                    zbuf_v, acc_sh.at[pl.ds(seg_base + kk * ZROWS, ZROWS)])
            if rem:
                pltpu.sync_copy(
                    zbuf_v.at[pl.ds(0, rem)],
                    acc_sh.at[pl.ds(seg_base + full * ZROWS, rem)])

        @pl.when(sid < NS - 1)
        def _():
            init_slab(SLAB)

        @pl.when(sid == NS - 1)
        def _():
            init_slab(SLAB_LAST)

        plsc.subcore_barrier()

        rows = (rows_a, rows_b)
        idx = (idx_a, idx_b)
        sem = (sem_a, sem_b)
        ssem = (ssem_a, ssem_b)

        def start_fetch(j, b):
            c = wid + j * NW
            base = pl.multiple_of(c * C, 8)

            @pl.when(c < NCHUNK)
            def _():
                pltpu.async_copy(x_hbm.at[pl.ds(base, C)], rows[b], sem[b])
                pltpu.async_copy(b_hbm.at[pl.ds(base, C)], idx[b], sem[b])

        def wait_fetch(j, b):
            c = wid + j * NW
            base = pl.multiple_of(c * C, 8)

            @pl.when(c < NCHUNK)
            def _():
                pltpu.make_async_copy(
                    x_hbm.at[pl.ds(base, C)], rows[b], sem[b]).wait()
                pltpu.make_async_copy(
                    b_hbm.at[pl.ds(base, C)], idx[b], sem[b]).wait()

        def start_scatter(j, b):
            c = wid + j * NW

            @pl.when(c < NCHUNK)
            def _():
                pltpu.async_copy(rows[b], acc_sh.at[idx[b]], ssem[b],
                                 add=True)

        def wait_scatter(j, b):
            c = wid + j * NW

            @pl.when(jnp.logical_and(c >= 0, c < NCHUNK))
            def _():
                pltpu.make_async_copy(
                    rows[b], acc_sh.at[idx[b]], ssem[b]).wait()

        start_fetch(0, 0)

        def body(t, carry):
            j0 = t * 2
            wait_fetch(j0, 0)
            start_scatter(j0, 0)
            wait_scatter(j0 - 1, 1)
            start_fetch(j0 + 1, 1)
            wait_fetch(j0 + 1, 1)
            start_scatter(j0 + 1, 1)
            wait_scatter(j0, 0)
            start_fetch(j0 + 2, 0)
            return carry

        lax.fori_loop(0, (ITERS + 1) // 2, body, 0)
        plsc.subcore_barrier()

        def dump_slab(nrows):
            full, rem = divmod(nrows, ZROWS)
            for kk in range(full):
                off = pl.multiple_of(seg_base + kk * ZROWS, 8)
                pltpu.sync_copy(acc_sh.at[pl.ds(off, ZROWS)],
                                out_hbm.at[cid, pl.ds(off, ZROWS)])
            if rem:
                off = pl.multiple_of(seg_base + full * ZROWS, 8)
                pltpu.sync_copy(acc_sh.at[pl.ds(off, rem)],
                                out_hbm.at[cid, pl.ds(off, rem)])

        @pl.when(sid < NS - 1)
        def _():
            dump_slab(SLAB)

        @pl.when(sid == NS - 1)
        def _():
            dump_slab(SLAB_LAST)

    return k(x, batch)


def _merge(p0, p1):
    def mk(a_ref, b_ref, o_ref):
        o_ref[...] = a_ref[...] + b_ref[...]

    return pl.pallas_call(
        mk,
        out_shape=jax.ShapeDtypeStruct((S, D), jnp.float32),
        grid=(10,),
        in_specs=[
            pl.BlockSpec((1000, D), lambda i: (i, 0)),
            pl.BlockSpec((1000, D), lambda i: (i, 0)),
        ],
        out_specs=pl.BlockSpec((1000, D), lambda i: (i, 0)),
    )(p0, p1)


@jax.jit
def kernel(x, batch):
    partials = _sc_partials(x, batch)
    return _merge(partials[0], partials[1])


# speedup vs baseline: 1.1078x; 1.1078x over previous
import functools

import jax
import jax.numpy as jnp
from jax import lax
from jax.experimental import pallas as pl
from jax.experimental.pallas import tpu as pltpu
from jax.experimental.pallas import tpu_sc as plsc

N = 320000
D = 128
S = 10000
C = 128
NCHUNK = N // C
NC = 2
NS = 16
NW = NC * NS
ITERS = (NCHUNK + NW - 1) // NW
SLAB = 632
SLAB_LAST = S - (NS - 1) * SLAB
ZROWS = 128


def _sc_partials(x, batch):
    mesh = plsc.VectorSubcoreMesh(core_axis_name="c", subcore_axis_name="s")

    @functools.partial(
        pl.kernel,
        mesh=mesh,
        out_type=jax.ShapeDtypeStruct((NC, S, D), jnp.float32),
        scratch_types=[
            pltpu.VMEM((C, D), jnp.float32),
            pltpu.VMEM((C, D), jnp.float32),
            pltpu.VMEM((C,), jnp.int32),
            pltpu.VMEM((C,), jnp.int32),
            pltpu.VMEM((ZROWS, D), jnp.float32),
            pltpu.VMEM_SHARED((S, D), jnp.float32),
            pltpu.SemaphoreType.DMA,
            pltpu.SemaphoreType.DMA,
        ],
    )
    def k(x_hbm, b_hbm, out_hbm, rows_a, rows_b, idx_a, idx_b, zbuf_v,
          acc_sh, sem_a, sem_b):
        cid = lax.axis_index("c")
        sid = lax.axis_index("s")
        wid = sid * NC + cid

        zeros16 = jnp.zeros((16,), jnp.float32)

        def zbody(i, carry):
            zbuf_v[i // 8, pl.ds((i % 8) * 16, 16)] = zeros16
            return carry

        lax.fori_loop(0, ZROWS * 8, zbody, 0)

        seg_base = pl.multiple_of(sid * SLAB, 8)

        def init_slab(nrows):
            full, rem = divmod(nrows, ZROWS)
            for kk in range(full):
                pltpu.sync_copy(
                    zbuf_v, acc_sh.at[pl.ds(seg_base + kk * ZROWS, ZROWS)])
            if rem:
                pltpu.sync_copy(
                    zbuf_v.at[pl.ds(0, rem)],
                    acc_sh.at[pl.ds(seg_base + full * ZROWS, rem)])

        @pl.when(sid < NS - 1)
        def _():
            init_slab(SLAB)

        @pl.when(sid == NS - 1)
        def _():
            init_slab(SLAB_LAST)

        plsc.subcore_barrier()

        rows = (rows_a, rows_b)
        idx = (idx_a, idx_b)
        sem = (sem_a, sem_b)

        def start_fetch(j, b):
            c = wid + j * NW
            base = pl.multiple_of(c * C, 8)

            @pl.when(c < NCHUNK)
            def _():
                pltpu.async_copy(x_hbm.at[pl.ds(base, C)], rows[b], sem[b])
                pltpu.async_copy(b_hbm.at[pl.ds(base, C)], idx[b], sem[b])

        def wait_and_scatter(j, b):
            c = wid + j * NW
            base = pl.multiple_of(c * C, 8)

            @pl.when(c < NCHUNK)
            def _():
                pltpu.make_async_copy(
                    x_hbm.at[pl.ds(base, C)], rows[b], sem[b]).wait()
                pltpu.make_async_copy(
                    b_hbm.at[pl.ds(base, C)], idx[b], sem[b]).wait()
                pltpu.sync_copy(rows[b], acc_sh.at[idx[b]], add=True)

        start_fetch(0, 0)
        start_fetch(1, 1)

        def body(t, carry):
            j0 = t * 2
            for b in range(2):
                wait_and_scatter(j0 + b, b)
                start_fetch(j0 + b + 2, b)
            return carry

        lax.fori_loop(0, (ITERS + 1) // 2, body, 0)
        plsc.subcore_barrier()

        def dump_slab(nrows):
            full, rem = divmod(nrows, ZROWS)
            for kk in range(full):
                off = pl.multiple_of(seg_base + kk * ZROWS, 8)
                pltpu.sync_copy(acc_sh.at[pl.ds(off, ZROWS)],
                                out_hbm.at[cid, pl.ds(off, ZROWS)])
            if rem:
                off = pl.multiple_of(seg_base + full * ZROWS, 8)
                pltpu.sync_copy(acc_sh.at[pl.ds(off, rem)],
                                out_hbm.at[cid, pl.ds(off, rem)])

        @pl.when(sid < NS - 1)
        def _():
            dump_slab(SLAB)

        @pl.when(sid == NS - 1)
        def _():
            dump_slab(SLAB_LAST)

    return k(x, batch)


def _merge(p0, p1):
    def mk(a_ref, b_ref, o_ref):
        o_ref[...] = a_ref[...] + b_ref[...]

    return pl.pallas_call(
        mk,
        out_shape=jax.ShapeDtypeStruct((S, D), jnp.float32),
        grid=(10,),
        in_specs=[
            pl.BlockSpec((1000, D), lambda i: (i, 0)),
            pl.BlockSpec((1000, D), lambda i: (i, 0)),
        ],
        out_specs=pl.BlockSpec((1000, D), lambda i: (i, 0)),
    )(p0, p1)


@jax.jit
def kernel(x, batch):
    partials = _sc_partials(x, batch)
    return _merge(partials[0], partials[1])
